# Initial kernel scaffold; baseline (speedup 1.0000x reference)
import jax
import jax.numpy as jnp
from jax.experimental import pallas as pl


def kernel(inputs, embedding, embedding_unnormalized):
    raise NotImplementedError("write your pallas kernel here")


# speedup vs baseline: 1.2945x; 1.2945x over previous
import functools

import jax
import jax.numpy as jnp
from jax import lax
from jax.experimental import pallas as pl

B, D, T = 32, 64, 576
K = 1024
N = B * T
BLK = 512
NBLK = N // BLK


def _vq_block(x_ref, emb_ref, embu_ref, quant_ref, idx_ref, loss_ref):
    x = x_ref[...]
    nrm = jnp.sqrt(jnp.sum(x * x, axis=1, keepdims=True))
    xn = x / jnp.maximum(nrm, 1e-12)
    emb = emb_ref[...]
    dist = lax.dot_general(
        xn, emb, (((1,), (1,)), ((), ())),
        preferred_element_type=jnp.float32)
    maxv = jnp.max(dist, axis=1, keepdims=True)
    kiota = lax.broadcasted_iota(jnp.int32, (BLK, K), 1)
    idx = jnp.min(jnp.where(dist == maxv, kiota, K), axis=1)
    oh = (kiota == idx[:, None]).astype(jnp.float32)
    quant = lax.dot_general(
        oh, emb, (((1,), (0,)), ((), ())),
        preferred_element_type=jnp.float32,
        precision=lax.Precision.HIGHEST)
    embu = embu_ref[...]
    nu = jnp.sqrt(jnp.sum(embu * embu, axis=1))
    nu_sel = jnp.sum(oh * nu[None, :], axis=1)
    gu = quant * nu_sel[:, None]
    d2 = x - gu
    quant_ref[...] = quant
    idx_ref[0, 0, :] = idx
    loss_ref[0, 0, :] = jnp.sum(d2 * d2, axis=0)


@jax.jit
def kernel(inputs, embedding, embedding_unnormalized):
    flat = jnp.transpose(inputs, (0, 2, 1)).reshape(N, D)
    quant, idx, loss = pl.pallas_call(
        _vq_block,
        grid=(NBLK,),
        in_specs=[
            pl.BlockSpec((BLK, D), lambda i: (i, 0)),
            pl.BlockSpec((K, D), lambda i: (0, 0)),
            pl.BlockSpec((K, D), lambda i: (0, 0)),
        ],
        out_specs=[
            pl.BlockSpec((BLK, D), lambda i: (i, 0)),
            pl.BlockSpec((1, 1, BLK), lambda i: (i, 0, 0)),
            pl.BlockSpec((1, 1, D), lambda i: (i, 0, 0)),
        ],
        out_shape=[
            jax.ShapeDtypeStruct((N, D), jnp.float32),
            jax.ShapeDtypeStruct((NBLK, 1, BLK), jnp.int32),
            jax.ShapeDtypeStruct((NBLK, 1, D), jnp.float32),
        ],
    )(flat, embedding, embedding_unnormalized)
    quantized = jnp.transpose(quant.reshape(B, T, D), (0, 2, 1))
    loss_val = jnp.sum(loss) / (N * D)
    return quantized, loss_val, loss_val, idx.reshape(N)


# speedup vs baseline: 1.8376x; 1.4196x over previous
import functools

import jax
import jax.numpy as jnp
from jax import lax
from jax.experimental import pallas as pl

B, D, T = 32, 64, 576
K = 1024
N = B * T


def _vq_block(x_ref, emb_ref, embu_ref, quant_ref, idx_ref, loss_ref):
    x = x_ref[0]
    nrm = jnp.sqrt(jnp.sum(x * x, axis=0, keepdims=True))
    xn = x / jnp.maximum(nrm, 1e-12)
    emb = emb_ref[...]
    dist = lax.dot_general(
        emb, xn, (((1,), (0,)), ((), ())),
        preferred_element_type=jnp.float32)
    maxv = jnp.max(dist, axis=0, keepdims=True)
    kiota = lax.broadcasted_iota(jnp.int32, (K, T), 0)
    idx = jnp.min(jnp.where(dist == maxv, kiota, K), axis=0)
    oh = (kiota == idx[None, :]).astype(jnp.float32)
    quant = lax.dot_general(
        emb, oh, (((0,), (0,)), ((), ())),
        preferred_element_type=jnp.float32,
        precision=lax.Precision.HIGHEST)
    embu = embu_ref[...]
    nu = jnp.sqrt(jnp.sum(embu * embu, axis=1))
    nu_sel = jnp.sum(oh * nu[:, None], axis=0)
    gu = quant * nu_sel[None, :]
    d2 = x - gu
    quant_ref[0] = quant
    idx_ref[0, 0, :] = idx
    loss_ref[0, 0, :] = jnp.sum(d2 * d2, axis=0)


@jax.jit
def kernel(inputs, embedding, embedding_unnormalized):
    quant, idx, loss = pl.pallas_call(
        _vq_block,
        grid=(B,),
        in_specs=[
            pl.BlockSpec((1, D, T), lambda i: (i, 0, 0)),
            pl.BlockSpec((K, D), lambda i: (0, 0)),
            pl.BlockSpec((K, D), lambda i: (0, 0)),
        ],
        out_specs=[
            pl.BlockSpec((1, D, T), lambda i: (i, 0, 0)),
            pl.BlockSpec((1, 1, T), lambda i: (i, 0, 0)),
            pl.BlockSpec((1, 1, T), lambda i: (i, 0, 0)),
        ],
        out_shape=[
            jax.ShapeDtypeStruct((B, D, T), jnp.float32),
            jax.ShapeDtypeStruct((B, 1, T), jnp.int32),
            jax.ShapeDtypeStruct((B, 1, T), jnp.float32),
        ],
    )(inputs, embedding, embedding_unnormalized)
    loss_val = jnp.sum(loss) / (N * D)
    return quant, loss_val, loss_val, idx.reshape(N)


# speedup vs baseline: 3.0023x; 1.6338x over previous
import functools

import jax
import jax.numpy as jnp
from jax import lax
from jax.experimental import pallas as pl
from jax.experimental.pallas import tpu as pltpu

B, D, T = 32, 64, 576
K = 1024
N = B * T


def _vq_block(x_ref, emb_ref, embu_ref, quant_ref, idx_ref, loss_ref,
              aug_ref, lo_ref):
    @pl.when(pl.program_id(0) == 0)
    def _prologue():
        embu = embu_ref[...]
        nu = jnp.sqrt(jnp.sum(embu * embu, axis=1))
        aug = jnp.concatenate(
            [emb_ref[...], jnp.broadcast_to(nu[:, None], (K, 128 - D))],
            axis=1)
        hi = aug.astype(jnp.bfloat16)
        aug_ref[...] = hi
        lo_ref[...] = (aug - hi.astype(jnp.float32)).astype(jnp.bfloat16)

    x = x_ref[0]
    nrm = jnp.sqrt(jnp.sum(x * x, axis=0, keepdims=True))
    xn = x / jnp.maximum(nrm, 1e-12)
    dist = lax.dot_general(
        emb_ref[...], xn, (((1,), (0,)), ((), ())),
        preferred_element_type=jnp.float32)
    maxv = jnp.max(dist, axis=0, keepdims=True)
    kiota = lax.broadcasted_iota(jnp.int32, (K, T), 0)
    idx = jnp.min(jnp.where(dist == maxv, kiota, K), axis=0)
    oh = (kiota == idx[None, :]).astype(jnp.bfloat16)
    q = (lax.dot_general(aug_ref[...], oh, (((0,), (0,)), ((), ())),
                         preferred_element_type=jnp.float32)
         + lax.dot_general(lo_ref[...], oh, (((0,), (0,)), ((), ())),
                           preferred_element_type=jnp.float32))
    quant = q[0:D, :]
    nu_sel = q[D:D + 1, :]
    gu = quant * nu_sel
    d2 = x - gu
    quant_ref[0] = quant
    idx_ref[0, 0, :] = idx
    loss_ref[0, 0, :] = jnp.sum(d2 * d2, axis=0)


@jax.jit
def kernel(inputs, embedding, embedding_unnormalized):
    quant, idx, loss = pl.pallas_call(
        _vq_block,
        grid=(B,),
        in_specs=[
            pl.BlockSpec((1, D, T), lambda i: (i, 0, 0)),
            pl.BlockSpec((K, D), lambda i: (0, 0)),
            pl.BlockSpec((K, D), lambda i: (0, 0)),
        ],
        out_specs=[
            pl.BlockSpec((1, D, T), lambda i: (i, 0, 0)),
            pl.BlockSpec((1, 1, T), lambda i: (i, 0, 0)),
            pl.BlockSpec((1, 1, T), lambda i: (i, 0, 0)),
        ],
        out_shape=[
            jax.ShapeDtypeStruct((B, D, T), jnp.float32),
            jax.ShapeDtypeStruct((B, 1, T), jnp.int32),
            jax.ShapeDtypeStruct((B, 1, T), jnp.float32),
        ],
        scratch_shapes=[
            pltpu.VMEM((K, 128), jnp.bfloat16),
            pltpu.VMEM((K, 128), jnp.bfloat16),
        ],
    )(inputs, embedding, embedding_unnormalized)
    loss_val = jnp.sum(loss) / (N * D)
    return quant, loss_val, loss_val, idx.reshape(N)


# speedup vs baseline: 3.3841x; 1.1272x over previous
import functools

import jax
import jax.numpy as jnp
from jax import lax
from jax.experimental import pallas as pl
from jax.experimental.pallas import tpu as pltpu

B, D, T = 32, 64, 576
K = 1024
N = B * T


def _vq_block(x_ref, emb_ref, embu_ref, quant_ref, idx_ref, loss_ref,
              aug_ref):
    @pl.when(pl.program_id(0) == 0)
    def _prologue():
        embu = embu_ref[...]
        nu = jnp.sqrt(jnp.sum(embu * embu, axis=1))
        aug = jnp.concatenate(
            [emb_ref[...], jnp.broadcast_to(nu[:, None], (K, 128 - D))],
            axis=1)
        aug_ref[...] = aug.astype(jnp.bfloat16)

    x = x_ref[0]
    nrm = jnp.sqrt(jnp.sum(x * x, axis=0, keepdims=True))
    xn = x / jnp.maximum(nrm, 1e-12)
    dist = lax.dot_general(
        emb_ref[...], xn, (((1,), (0,)), ((), ())),
        preferred_element_type=jnp.float32)
    maxv = jnp.max(dist, axis=0, keepdims=True)
    kiota = lax.broadcasted_iota(jnp.int32, (K, T), 0)
    idx = jnp.min(jnp.where(dist == maxv, kiota, K), axis=0)
    oh = (kiota == idx[None, :]).astype(jnp.bfloat16)
    q = lax.dot_general(aug_ref[...], oh, (((0,), (0,)), ((), ())),
                        preferred_element_type=jnp.float32)
    quant = q[0:D, :]
    nu_sel = q[D:D + 1, :]
    gu = quant * nu_sel
    d2 = x - gu
    quant_ref[0] = quant
    idx_ref[0, 0, :] = idx
    loss_ref[0, 0, :] = jnp.sum(d2 * d2, axis=0)


@jax.jit
def kernel(inputs, embedding, embedding_unnormalized):
    quant, idx, loss = pl.pallas_call(
        _vq_block,
        grid=(B,),
        in_specs=[
            pl.BlockSpec((1, D, T), lambda i: (i, 0, 0)),
            pl.BlockSpec((K, D), lambda i: (0, 0)),
            pl.BlockSpec((K, D), lambda i: (0, 0)),
        ],
        out_specs=[
            pl.BlockSpec((1, D, T), lambda i: (i, 0, 0)),
            pl.BlockSpec((1, 1, T), lambda i: (i, 0, 0)),
            pl.BlockSpec((1, 1, T), lambda i: (i, 0, 0)),
        ],
        out_shape=[
            jax.ShapeDtypeStruct((B, D, T), jnp.float32),
            jax.ShapeDtypeStruct((B, 1, T), jnp.int32),
            jax.ShapeDtypeStruct((B, 1, T), jnp.float32),
        ],
        scratch_shapes=[
            pltpu.VMEM((K, 128), jnp.bfloat16),
        ],
    )(inputs, embedding, embedding_unnormalized)
    loss_val = jnp.sum(loss) / (N * D)
    return quant, loss_val, loss_val, idx.reshape(N)


# speedup vs baseline: 3.7508x; 1.1084x over previous
import functools

import jax
import jax.numpy as jnp
from jax import lax
from jax.experimental import pallas as pl
from jax.experimental.pallas import tpu as pltpu

B, D, T = 32, 64, 576
K = 1024
N = B * T


def _vq_block(x_ref, emb_ref, embu_ref, quant_ref, idx_ref, loss_ref,
              aug_ref, q_ref, acc_ref):
    i = pl.program_id(0)

    @pl.when(i == 0)
    def _prologue():
        embu = embu_ref[...]
        nu = jnp.sqrt(jnp.sum(embu * embu, axis=1, keepdims=True))
        kvec = lax.broadcasted_iota(jnp.int32, (K, 1), 0)
        khi = (kvec & ~7).astype(jnp.float32)
        klo = (kvec & 7).astype(jnp.float32)
        ones = jnp.ones((K, 1), jnp.float32)
        aug = jnp.concatenate(
            [emb_ref[...], nu, khi, klo, ones,
             jnp.zeros((K, 128 - D - 4), jnp.float32)], axis=1)
        aug_ref[...] = aug.astype(jnp.bfloat16)
        acc_ref[0] = 0.0

    x = x_ref[0]
    nrm = jnp.sqrt(jnp.sum(x * x, axis=0, keepdims=True))
    xn = x / jnp.maximum(nrm, 1e-12)
    dist = lax.dot_general(
        emb_ref[...], xn, (((1,), (0,)), ((), ())),
        preferred_element_type=jnp.float32)
    maxv = jnp.max(dist, axis=0, keepdims=True)
    eq = dist >= maxv
    q_ref[...] = lax.dot_general(
        aug_ref[...], eq.astype(jnp.bfloat16), (((0,), (0,)), ((), ())),
        preferred_element_type=jnp.float32)
    tie = jnp.max(q_ref[D + 3, :]) > 1.5

    @pl.when(tie)
    def _exact_tiebreak():
        kiota = lax.broadcasted_iota(jnp.int32, (K, T), 0)
        idx_e = jnp.min(jnp.where(eq, kiota, K), axis=0)
        oh = (kiota == idx_e[None, :]).astype(jnp.bfloat16)
        q_ref[...] = lax.dot_general(
            aug_ref[...], oh, (((0,), (0,)), ((), ())),
            preferred_element_type=jnp.float32)

    q = q_ref[...]
    quant = q[0:D, :]
    nu_sel = q[D:D + 1, :]
    idx = (q[D + 1, :] + q[D + 2, :]).astype(jnp.int32)
    gu = quant * nu_sel
    d2 = x - gu
    quant_ref[0] = quant
    idx_ref[0, 0, :] = idx
    acc_ref[0] += jnp.sum(d2 * d2)

    @pl.when(i == B - 1)
    def _epilogue():
        loss_ref[0] = acc_ref[0] / (N * D)


@jax.jit
def kernel(inputs, embedding, embedding_unnormalized):
    quant, idx, loss = pl.pallas_call(
        _vq_block,
        grid=(B,),
        in_specs=[
            pl.BlockSpec((1, D, T), lambda i: (i, 0, 0)),
            pl.BlockSpec((K, D), lambda i: (0, 0)),
            pl.BlockSpec((K, D), lambda i: (0, 0)),
        ],
        out_specs=[
            pl.BlockSpec((1, D, T), lambda i: (i, 0, 0)),
            pl.BlockSpec((1, 1, T), lambda i: (i, 0, 0)),
            pl.BlockSpec(memory_space=pltpu.SMEM),
        ],
        out_shape=[
            jax.ShapeDtypeStruct((B, D, T), jnp.float32),
            jax.ShapeDtypeStruct((B, 1, T), jnp.int32),
            jax.ShapeDtypeStruct((1,), jnp.float32),
        ],
        scratch_shapes=[
            pltpu.VMEM((K, 128), jnp.bfloat16),
            pltpu.VMEM((128, T), jnp.float32),
            pltpu.SMEM((1,), jnp.float32),
        ],
    )(inputs, embedding, embedding_unnormalized)
    loss_val = loss[0]
    return quant, loss_val, loss_val, idx.reshape(N)


# speedup vs baseline: 5.4125x; 1.4430x over previous
import functools

import jax
import jax.numpy as jnp
from jax import lax
from jax.experimental import pallas as pl
from jax.experimental.pallas import tpu as pltpu

B, D, T = 32, 64, 576
K = 1024
N = B * T
BB = 2
W = BB * T
NBLK = B // BB


def _vq_block(x_ref, emb_ref, embu_ref, quant_ref, idx_ref, loss_ref,
              aug_ref, q_ref, acc_ref):
    i = pl.program_id(0)

    @pl.when(i == 0)
    def _prologue():
        embu = embu_ref[...]
        nu = jnp.sqrt(jnp.sum(embu * embu, axis=1, keepdims=True))
        kvec = lax.broadcasted_iota(jnp.int32, (K, 1), 0)
        khi = (kvec & ~7).astype(jnp.float32)
        klo = (kvec & 7).astype(jnp.float32)
        ones = jnp.ones((K, 1), jnp.float32)
        aug = jnp.concatenate(
            [emb_ref[...], nu, khi, klo, ones,
             jnp.zeros((K, 128 - D - 4), jnp.float32)], axis=1)
        aug_ref[...] = aug.astype(jnp.bfloat16)
        acc_ref[0] = 0.0

    x = jnp.concatenate([x_ref[0], x_ref[1]], axis=1)
    nrm = jnp.sqrt(jnp.sum(x * x, axis=0, keepdims=True))
    xn = x / jnp.maximum(nrm, 1e-12)
    dist = lax.dot_general(
        emb_ref[...], xn, (((1,), (0,)), ((), ())),
        preferred_element_type=jnp.float32)
    maxv = jnp.max(dist, axis=0, keepdims=True)
    eq = dist >= maxv
    q_ref[...] = lax.dot_general(
        aug_ref[...], eq.astype(jnp.bfloat16), (((0,), (0,)), ((), ())),
        preferred_element_type=jnp.float32)
    tie = jnp.max(q_ref[D + 3, :]) > 1.5

    @pl.when(tie)
    def _exact_tiebreak():
        kiota = lax.broadcasted_iota(jnp.int32, (K, W), 0)
        idx_e = jnp.min(jnp.where(eq, kiota, K), axis=0)
        oh = (kiota == idx_e[None, :]).astype(jnp.bfloat16)
        q_ref[...] = lax.dot_general(
            aug_ref[...], oh, (((0,), (0,)), ((), ())),
            preferred_element_type=jnp.float32)

    q = q_ref[...]
    quant = q[0:D, :]
    nu_sel = q[D:D + 1, :]
    idx = (q[D + 1, :] + q[D + 2, :]).astype(jnp.int32)
    gu = quant * nu_sel
    d2 = x - gu
    quant_ref[0] = quant[:, 0:T]
    quant_ref[1] = quant[:, T:W]
    idx_ref[0, 0, :] = idx
    acc_ref[0] += jnp.sum(d2 * d2)

    @pl.when(i == NBLK - 1)
    def _epilogue():
        loss_ref[0] = acc_ref[0] / (N * D)


@jax.jit
def kernel(inputs, embedding, embedding_unnormalized):
    quant, idx, loss = pl.pallas_call(
        _vq_block,
        grid=(NBLK,),
        in_specs=[
            pl.BlockSpec((BB, D, T), lambda i: (i, 0, 0)),
            pl.BlockSpec((K, D), lambda i: (0, 0)),
            pl.BlockSpec((K, D), lambda i: (0, 0)),
        ],
        out_specs=[
            pl.BlockSpec((BB, D, T), lambda i: (i, 0, 0)),
            pl.BlockSpec((1, 1, W), lambda i: (i, 0, 0)),
            pl.BlockSpec(memory_space=pltpu.SMEM),
        ],
        out_shape=[
            jax.ShapeDtypeStruct((B, D, T), jnp.float32),
            jax.ShapeDtypeStruct((NBLK, 1, W), jnp.int32),
            jax.ShapeDtypeStruct((1,), jnp.float32),
        ],
        scratch_shapes=[
            pltpu.VMEM((K, 128), jnp.bfloat16),
            pltpu.VMEM((128, W), jnp.float32),
            pltpu.SMEM((1,), jnp.float32),
        ],
    )(inputs, embedding, embedding_unnormalized)
    loss_val = loss[0]
    return quant, loss_val, loss_val, idx.reshape(N)


# speedup vs baseline: 6.0403x; 1.1160x over previous
import functools

import jax
import jax.numpy as jnp
from jax import lax
from jax.experimental import pallas as pl
from jax.experimental.pallas import tpu as pltpu

B, D, T = 32, 64, 576
K = 1024
N = B * T
BB = 4
W = BB * T
NBLK = B // BB


def _vq_block(x_ref, emb_ref, embu_ref, quant_ref, idx_ref, loss_ref,
              aug_ref, q_ref, acc_ref):
    i = pl.program_id(0)

    @pl.when(i == 0)
    def _prologue():
        embu = embu_ref[...]
        nu = jnp.sqrt(jnp.sum(embu * embu, axis=1, keepdims=True))
        kvec = lax.broadcasted_iota(jnp.int32, (K, 1), 0)
        khi = (kvec & ~7).astype(jnp.float32)
        klo = (kvec & 7).astype(jnp.float32)
        ones = jnp.ones((K, 1), jnp.float32)
        aug = jnp.concatenate(
            [emb_ref[...], nu, khi, klo, ones,
             jnp.zeros((K, 128 - D - 4), jnp.float32)], axis=1)
        aug_ref[...] = aug.astype(jnp.bfloat16)
        acc_ref[0] = 0.0

    x = jnp.concatenate([x_ref[b] for b in range(BB)], axis=1)
    nrm = jnp.sqrt(jnp.sum(x * x, axis=0, keepdims=True))
    xn = x / jnp.maximum(nrm, 1e-12)
    dist = lax.dot_general(
        emb_ref[...], xn, (((1,), (0,)), ((), ())),
        preferred_element_type=jnp.float32)
    maxv = jnp.max(dist, axis=0, keepdims=True)
    eq = dist >= maxv
    q_ref[...] = lax.dot_general(
        aug_ref[...], eq.astype(jnp.bfloat16), (((0,), (0,)), ((), ())),
        preferred_element_type=jnp.float32)
    tie = jnp.max(q_ref[D + 3, :]) > 1.5

    @pl.when(tie)
    def _exact_tiebreak():
        kiota = lax.broadcasted_iota(jnp.int32, (K, W), 0)
        idx_e = jnp.min(jnp.where(eq, kiota, K), axis=0)
        oh = (kiota == idx_e[None, :]).astype(jnp.bfloat16)
        q_ref[...] = lax.dot_general(
            aug_ref[...], oh, (((0,), (0,)), ((), ())),
            preferred_element_type=jnp.float32)

    q = q_ref[...]
    quant = q[0:D, :]
    nu_sel = q[D:D + 1, :]
    idx = (q[D + 1, :] + q[D + 2, :]).astype(jnp.int32)
    gu = quant * nu_sel
    d2 = x - gu
    for b in range(BB):
        quant_ref[b] = quant[:, b * T:(b + 1) * T]
    idx_ref[0, 0, :] = idx
    acc_ref[0] += jnp.sum(d2 * d2)

    @pl.when(i == NBLK - 1)
    def _epilogue():
        loss_ref[0] = acc_ref[0] / (N * D)


@jax.jit
def kernel(inputs, embedding, embedding_unnormalized):
    quant, idx, loss = pl.pallas_call(
        _vq_block,
        grid=(NBLK,),
        in_specs=[
            pl.BlockSpec((BB, D, T), lambda i: (i, 0, 0)),
            pl.BlockSpec((K, D), lambda i: (0, 0)),
            pl.BlockSpec((K, D), lambda i: (0, 0)),
        ],
        out_specs=[
            pl.BlockSpec((BB, D, T), lambda i: (i, 0, 0)),
            pl.BlockSpec((1, 1, W), lambda i: (i, 0, 0)),
            pl.BlockSpec(memory_space=pltpu.SMEM),
        ],
        out_shape=[
            jax.ShapeDtypeStruct((B, D, T), jnp.float32),
            jax.ShapeDtypeStruct((NBLK, 1, W), jnp.int32),
            jax.ShapeDtypeStruct((1,), jnp.float32),
        ],
        scratch_shapes=[
            pltpu.VMEM((K, 128), jnp.bfloat16),
            pltpu.VMEM((128, W), jnp.float32),
            pltpu.SMEM((1,), jnp.float32),
        ],
    )(inputs, embedding, embedding_unnormalized)
    loss_val = loss[0]
    return quant, loss_val, loss_val, idx.reshape(N)


# speedup vs baseline: 6.2556x; 1.0356x over previous
import functools

import jax
import jax.numpy as jnp
from jax import lax
from jax.experimental import pallas as pl
from jax.experimental.pallas import tpu as pltpu

B, D, T = 32, 64, 576
K = 1024
N = B * T
BB = 8
W = BB * T
NBLK = B // BB


def _vq_block(x_ref, emb_ref, embu_ref, quant_ref, idx_ref, loss_ref,
              aug_ref, q_ref, acc_ref):
    i = pl.program_id(0)

    @pl.when(i == 0)
    def _prologue():
        embu = embu_ref[...]
        nu = jnp.sqrt(jnp.sum(embu * embu, axis=1, keepdims=True))
        kvec = lax.broadcasted_iota(jnp.int32, (K, 1), 0)
        khi = (kvec & ~7).astype(jnp.float32)
        klo = (kvec & 7).astype(jnp.float32)
        ones = jnp.ones((K, 1), jnp.float32)
        aug = jnp.concatenate(
            [emb_ref[...], nu, khi, klo, ones,
             jnp.zeros((K, 128 - D - 4), jnp.float32)], axis=1)
        aug_ref[...] = aug.astype(jnp.bfloat16)
        acc_ref[0] = 0.0

    x = jnp.concatenate([x_ref[b] for b in range(BB)], axis=1)
    nrm = jnp.sqrt(jnp.sum(x * x, axis=0, keepdims=True))
    xn = x / jnp.maximum(nrm, 1e-12)
    dist = lax.dot_general(
        emb_ref[...], xn, (((1,), (0,)), ((), ())),
        preferred_element_type=jnp.float32)
    maxv = jnp.max(dist, axis=0, keepdims=True)
    eq = dist >= maxv
    q_ref[...] = lax.dot_general(
        aug_ref[...], eq.astype(jnp.bfloat16), (((0,), (0,)), ((), ())),
        preferred_element_type=jnp.float32)
    tie = jnp.max(q_ref[D + 3, :]) > 1.5

    @pl.when(tie)
    def _exact_tiebreak():
        kiota = lax.broadcasted_iota(jnp.int32, (K, W), 0)
        idx_e = jnp.min(jnp.where(eq, kiota, K), axis=0)
        oh = (kiota == idx_e[None, :]).astype(jnp.bfloat16)
        q_ref[...] = lax.dot_general(
            aug_ref[...], oh, (((0,), (0,)), ((), ())),
            preferred_element_type=jnp.float32)

    q = q_ref[...]
    quant = q[0:D, :]
    nu_sel = q[D:D + 1, :]
    idx = (q[D + 1, :] + q[D + 2, :]).astype(jnp.int32)
    gu = quant * nu_sel
    d2 = x - gu
    for b in range(BB):
        quant_ref[b] = quant[:, b * T:(b + 1) * T]
    idx_ref[0, 0, :] = idx
    acc_ref[0] += jnp.sum(d2 * d2)

    @pl.when(i == NBLK - 1)
    def _epilogue():
        loss_ref[0] = acc_ref[0] / (N * D)


@jax.jit
def kernel(inputs, embedding, embedding_unnormalized):
    quant, idx, loss = pl.pallas_call(
        _vq_block,
        grid=(NBLK,),
        in_specs=[
            pl.BlockSpec((BB, D, T), lambda i: (i, 0, 0)),
            pl.BlockSpec((K, D), lambda i: (0, 0)),
            pl.BlockSpec((K, D), lambda i: (0, 0)),
        ],
        out_specs=[
            pl.BlockSpec((BB, D, T), lambda i: (i, 0, 0)),
            pl.BlockSpec((1, 1, W), lambda i: (i, 0, 0)),
            pl.BlockSpec(memory_space=pltpu.SMEM),
        ],
        out_shape=[
            jax.ShapeDtypeStruct((B, D, T), jnp.float32),
            jax.ShapeDtypeStruct((NBLK, 1, W), jnp.int32),
            jax.ShapeDtypeStruct((1,), jnp.float32),
        ],
        scratch_shapes=[
            pltpu.VMEM((K, 128), jnp.bfloat16),
            pltpu.VMEM((128, W), jnp.float32),
            pltpu.SMEM((1,), jnp.float32),
        ],
    )(inputs, embedding, embedding_unnormalized)
    loss_val = loss[0]
    return quant, loss_val, loss_val, idx.reshape(N)
